# baseline (speedup 1.0000x reference)
import functools

import jax
import jax.numpy as jnp
from jax import lax
from jax.experimental import pallas as pl
from jax.experimental.pallas import tpu as pltpu
from jax.experimental.pallas import tpu_sc as plsc

N = 10000
E = 320000
D = 128
G = 64

NC, NS = 2, 16
NW = NC * NS
CH = 128
EPW = E // NW
NCH = -(-EPW // CH)
EPW_PAD = NCH * CH
ROWS_PT = 632
N_PAD = NS * ROWS_PT

_R = 2000
_NG = N // _R


def _leaky(v):
    return jnp.where(v >= 0, v, 0.01 * v)


def _sc_agg(h, src_p, dst_p, zrows):
    mesh = plsc.VectorSubcoreMesh(
        core_axis_name="c", subcore_axis_name="s",
        num_cores=NC, num_subcores=NS)

    @functools.partial(
        pl.kernel,
        out_type=jax.ShapeDtypeStruct((NC, N_PAD, D), jnp.float32),
        mesh=mesh,
        scratch_types=[
            pltpu.VMEM((NCH, CH), jnp.int32),
            pltpu.VMEM((NCH, CH), jnp.int32),
            pltpu.VMEM((CH, D), jnp.float32),
            pltpu.VMEM_SHARED((N_PAD, D), jnp.float32),
            pltpu.SemaphoreType.DMA,
        ],
    )
    def k(h_hbm, src_hbm, dst_hbm, z_hbm, out_hbm, sidx, didx, rows, agg, sem):
        c = lax.axis_index("c")
        s = lax.axis_index("s")
        wid = s * NC + c
        pltpu.sync_copy(z_hbm, agg.at[pl.ds(s * ROWS_PT, ROWS_PT)])
        pltpu.sync_copy(src_hbm.at[wid], sidx)
        pltpu.sync_copy(dst_hbm.at[wid], didx)
        plsc.subcore_barrier()

        def body(j, carry):
            pltpu.async_copy(h_hbm.at[sidx.at[j]], rows, sem).wait()
            pltpu.sync_copy(rows, agg.at[didx.at[j]], add=True)
            return carry

        lax.fori_loop(0, NCH, body, 0)
        plsc.subcore_barrier()
        pltpu.sync_copy(agg.at[pl.ds(s * ROWS_PT, ROWS_PT)],
                        out_hbm.at[c].at[pl.ds(s * ROWS_PT, ROWS_PT)])

    return k(h, src_p, dst_p, zrows)


def _tc_proj(x, W0, b0, W1, b1, roW0, rb0):
    def body(x_ref, w0_ref, b0_ref, w1_ref, b1_ref, rw_ref, rb_ref,
             o_ref, rt_ref):
        h = _leaky(lax.dot_general(x_ref[...], w0_ref[...],
                                   (((1,), (1,)), ((), ())),
                                   preferred_element_type=jnp.float32)
                   + b0_ref[...])
        h = _leaky(lax.dot_general(h, w1_ref[...],
                                   (((1,), (1,)), ((), ())),
                                   preferred_element_type=jnp.float32)
                   + b1_ref[...])
        o_ref[...] = h
        rt_ref[...] = lax.dot_general(h, rw_ref[...], (((1,), (1,)), ((), ())),
                                      preferred_element_type=jnp.float32
                                      ) + rb_ref[...]

    return pl.pallas_call(
        body,
        grid=(_NG,),
        in_specs=[pl.BlockSpec((_R, D), lambda i: (i, 0)),
                  pl.BlockSpec((D, D), lambda i: (0, 0)),
                  pl.BlockSpec((1, D), lambda i: (0, 0)),
                  pl.BlockSpec((D, D), lambda i: (0, 0)),
                  pl.BlockSpec((1, D), lambda i: (0, 0)),
                  pl.BlockSpec((D, D), lambda i: (0, 0)),
                  pl.BlockSpec((1, D), lambda i: (0, 0))],
        out_specs=[pl.BlockSpec((_R, D), lambda i: (i, 0)),
                   pl.BlockSpec((_R, D), lambda i: (i, 0))],
        out_shape=[jax.ShapeDtypeStruct((N, D), jnp.float32),
                   jax.ShapeDtypeStruct((N, D), jnp.float32)],
    )(x, W0, b0.reshape(1, D), W1, b1.reshape(1, D), roW0, rb0.reshape(1, D))


def _tc_layer0(parts, root, resid, rW, lg, lb, roW1, rb1):
    def body(a_ref, b_ref, rt_ref, r_ref, rw_ref, lg_ref, lb_ref,
             rw1_ref, rb1_ref, o_ref, rt1_ref):
        agg = a_ref[0] + b_ref[0]
        new = lax.dot_general(agg, rw_ref[...], (((1,), (1,)), ((), ())),
                              preferred_element_type=jnp.float32) + rt_ref[...]
        mu = jnp.mean(new, axis=-1, keepdims=True)
        var = jnp.mean((new - mu) ** 2, axis=-1, keepdims=True)
        new = (new - mu) * lax.rsqrt(var + 1e-5) * lg_ref[...] + lb_ref[...]
        h1 = _leaky(new) + r_ref[...]
        o_ref[...] = h1
        rt1_ref[...] = lax.dot_general(h1, rw1_ref[...],
                                       (((1,), (1,)), ((), ())),
                                       preferred_element_type=jnp.float32
                                       ) + rb1_ref[...]

    return pl.pallas_call(
        body,
        grid=(_NG,),
        in_specs=[pl.BlockSpec((1, _R, D), lambda i: (0, i, 0)),
                  pl.BlockSpec((1, _R, D), lambda i: (1, i, 0)),
                  pl.BlockSpec((_R, D), lambda i: (i, 0)),
                  pl.BlockSpec((_R, D), lambda i: (i, 0)),
                  pl.BlockSpec((D, D), lambda i: (0, 0)),
                  pl.BlockSpec((1, D), lambda i: (0, 0)),
                  pl.BlockSpec((1, D), lambda i: (0, 0)),
                  pl.BlockSpec((D, D), lambda i: (0, 0)),
                  pl.BlockSpec((1, D), lambda i: (0, 0))],
        out_specs=[pl.BlockSpec((_R, D), lambda i: (i, 0)),
                   pl.BlockSpec((_R, D), lambda i: (i, 0))],
        out_shape=[jax.ShapeDtypeStruct((N, D), jnp.float32),
                   jax.ShapeDtypeStruct((N, D), jnp.float32)],
    )(parts, parts, root, resid, rW, lg.reshape(1, D), lb.reshape(1, D),
      roW1, rb1.reshape(1, D))


def _tc_layer1_pool(parts, root, resid, rW, lg, lb, batch2d):
    def body(a_ref, b_ref, rt_ref, r_ref, rw_ref, lg_ref, lb_ref, bt_ref,
             o_ref, s_sum, s_cnt):
        i = pl.program_id(0)

        @pl.when(i == 0)
        def _():
            s_sum[...] = jnp.zeros_like(s_sum)
            s_cnt[...] = jnp.zeros_like(s_cnt)

        agg = a_ref[0] + b_ref[0]
        new = lax.dot_general(agg, rw_ref[...], (((1,), (1,)), ((), ())),
                              preferred_element_type=jnp.float32) + rt_ref[...]
        mu = jnp.mean(new, axis=-1, keepdims=True)
        var = jnp.mean((new - mu) ** 2, axis=-1, keepdims=True)
        new = (new - mu) * lax.rsqrt(var + 1e-5) * lg_ref[...] + lb_ref[...]
        h2 = _leaky(new) + r_ref[...]

        oh = (bt_ref[...] == lax.broadcasted_iota(jnp.int32, (_R, G), 1)
              ).astype(jnp.float32)
        s_sum[...] += lax.dot_general(oh, h2, (((0,), (0,)), ((), ())),
                                      preferred_element_type=jnp.float32)
        s_cnt[...] += lax.dot_general(oh, jnp.ones((_R, 1), jnp.float32),
                                      (((0,), (0,)), ((), ())),
                                      preferred_element_type=jnp.float32)

        @pl.when(i == _NG - 1)
        def _():
            o_ref[...] = s_sum[...] / jnp.maximum(s_cnt[...], 1.0)

    return pl.pallas_call(
        body,
        grid=(_NG,),
        in_specs=[pl.BlockSpec((1, _R, D), lambda i: (0, i, 0)),
                  pl.BlockSpec((1, _R, D), lambda i: (1, i, 0)),
                  pl.BlockSpec((_R, D), lambda i: (i, 0)),
                  pl.BlockSpec((_R, D), lambda i: (i, 0)),
                  pl.BlockSpec((D, D), lambda i: (0, 0)),
                  pl.BlockSpec((1, D), lambda i: (0, 0)),
                  pl.BlockSpec((1, D), lambda i: (0, 0)),
                  pl.BlockSpec((_R, 1), lambda i: (i, 0))],
        out_specs=pl.BlockSpec((G, D), lambda i: (0, 0)),
        out_shape=jax.ShapeDtypeStruct((G, D), jnp.float32),
        scratch_shapes=[pltpu.VMEM((G, D), jnp.float32),
                        pltpu.VMEM((G, 1), jnp.float32)],
    )(parts, parts, root, resid, rW, lg.reshape(1, D), lb.reshape(1, D),
      batch2d)


def kernel(x, edge_index, batch, proj_W0, proj_b0, proj_W1, proj_b1,
           rel_W0, rel_b0, root_W0, ln_g0, ln_b0,
           rel_W1, rel_b1, root_W1, ln_g1, ln_b1):
    x = x.astype(jnp.float32)
    src, dst = edge_index[0], edge_index[1]
    pad = NW * EPW_PAD - E
    src_p = jnp.concatenate([src, jnp.zeros((pad,), jnp.int32)]
                            ).reshape(NW, NCH, CH)
    dst_p = jnp.concatenate([dst, jnp.full((pad,), N, jnp.int32)]
                            ).reshape(NW, NCH, CH)
    zrows = jnp.zeros((ROWS_PT, D), jnp.float32)

    h, root0 = _tc_proj(x, proj_W0, proj_b0, proj_W1, proj_b1,
                        root_W0, rel_b0)
    parts = _sc_agg(h, src_p, dst_p, zrows)
    h1, root1 = _tc_layer0(parts, root0, x, rel_W0, ln_g0, ln_b0,
                           root_W1, rel_b1)
    parts = _sc_agg(h1, src_p, dst_p, zrows)
    return _tc_layer1_pool(parts, root1, h1, rel_W1, ln_g1, ln_b1,
                           batch.reshape(N, 1))


# speedup vs baseline: 1.0190x; 1.0190x over previous
import functools

import jax
import jax.numpy as jnp
from jax import lax
from jax.experimental import pallas as pl
from jax.experimental.pallas import tpu as pltpu
from jax.experimental.pallas import tpu_sc as plsc

N = 10000
E = 320000
D = 128
G = 64

NC, NS = 2, 16
NW = NC * NS
CH = 128
EPW = E // NW
NCH = -(-EPW // CH)
EPW_PAD = NCH * CH
ROWS_PT = 632
N_PAD = NS * ROWS_PT

_R = 2000
_NG = N // _R


def _leaky(v):
    return jnp.where(v >= 0, v, 0.01 * v)


def _sc_agg(h, src_p, dst_p, zrows):
    mesh = plsc.VectorSubcoreMesh(
        core_axis_name="c", subcore_axis_name="s",
        num_cores=NC, num_subcores=NS)

    @functools.partial(
        pl.kernel,
        out_type=jax.ShapeDtypeStruct((NC, N_PAD, D), jnp.float32),
        mesh=mesh,
        scratch_types=[
            pltpu.VMEM((NCH, CH), jnp.int32),
            pltpu.VMEM((NCH, CH), jnp.int32),
            pltpu.VMEM((CH, D), jnp.float32),
            pltpu.VMEM_SHARED((N_PAD, D), jnp.float32),
            pltpu.SemaphoreType.DMA,
        ],
    )
    def k(h_hbm, src_hbm, dst_hbm, z_hbm, out_hbm, sidx, didx, rows, agg, sem):
        c = lax.axis_index("c")
        s = lax.axis_index("s")
        wid = s * NC + c
        pltpu.sync_copy(z_hbm, agg.at[pl.ds(s * ROWS_PT, ROWS_PT)])
        pltpu.sync_copy(src_hbm.at[wid], sidx)
        pltpu.sync_copy(dst_hbm.at[wid], didx)
        plsc.subcore_barrier()

        def body(j, carry):
            pltpu.async_copy(h_hbm.at[sidx.at[j]], rows, sem).wait()
            pltpu.sync_copy(rows, agg.at[didx.at[j]], add=True)
            return carry

        lax.fori_loop(0, NCH, body, 0)
        plsc.subcore_barrier()
        pltpu.sync_copy(agg.at[pl.ds(s * ROWS_PT, ROWS_PT)],
                        out_hbm.at[c].at[pl.ds(s * ROWS_PT, ROWS_PT)])

    return k(h, src_p, dst_p, zrows)


def _tc_proj(x, W0, b0, W1, b1):
    def body(x_ref, w0_ref, b0_ref, w1_ref, b1_ref, o_ref):
        h = _leaky(lax.dot_general(x_ref[...], w0_ref[...],
                                   (((1,), (1,)), ((), ())),
                                   preferred_element_type=jnp.float32)
                   + b0_ref[...])
        o_ref[...] = _leaky(lax.dot_general(h, w1_ref[...],
                                            (((1,), (1,)), ((), ())),
                                            preferred_element_type=jnp.float32)
                            + b1_ref[...])

    return pl.pallas_call(
        body,
        grid=(_NG,),
        in_specs=[pl.BlockSpec((_R, D), lambda i: (i, 0)),
                  pl.BlockSpec((D, D), lambda i: (0, 0)),
                  pl.BlockSpec((1, D), lambda i: (0, 0)),
                  pl.BlockSpec((D, D), lambda i: (0, 0)),
                  pl.BlockSpec((1, D), lambda i: (0, 0))],
        out_specs=pl.BlockSpec((_R, D), lambda i: (i, 0)),
        out_shape=jax.ShapeDtypeStruct((N, D), jnp.float32),
    )(x, W0, b0.reshape(1, D), W1, b1.reshape(1, D))


def _tc_root(h, roW, rb):
    def body(h_ref, w_ref, b_ref, o_ref):
        o_ref[...] = lax.dot_general(h_ref[...], w_ref[...],
                                     (((1,), (1,)), ((), ())),
                                     preferred_element_type=jnp.float32
                                     ) + b_ref[...]

    return pl.pallas_call(
        body,
        grid=(_NG,),
        in_specs=[pl.BlockSpec((_R, D), lambda i: (i, 0)),
                  pl.BlockSpec((D, D), lambda i: (0, 0)),
                  pl.BlockSpec((1, D), lambda i: (0, 0))],
        out_specs=pl.BlockSpec((_R, D), lambda i: (i, 0)),
        out_shape=jax.ShapeDtypeStruct((N, D), jnp.float32),
    )(h, roW, rb.reshape(1, D))


def _tc_layer(parts, root, resid, rW, lg, lb):
    def body(a_ref, b_ref, rt_ref, r_ref, rw_ref, lg_ref, lb_ref, o_ref):
        agg = a_ref[0] + b_ref[0]
        new = lax.dot_general(agg, rw_ref[...], (((1,), (1,)), ((), ())),
                              preferred_element_type=jnp.float32) + rt_ref[...]
        mu = jnp.mean(new, axis=-1, keepdims=True)
        var = jnp.mean((new - mu) ** 2, axis=-1, keepdims=True)
        new = (new - mu) * lax.rsqrt(var + 1e-5) * lg_ref[...] + lb_ref[...]
        o_ref[...] = _leaky(new) + r_ref[...]

    return pl.pallas_call(
        body,
        grid=(_NG,),
        in_specs=[pl.BlockSpec((1, _R, D), lambda i: (0, i, 0)),
                  pl.BlockSpec((1, _R, D), lambda i: (1, i, 0)),
                  pl.BlockSpec((_R, D), lambda i: (i, 0)),
                  pl.BlockSpec((_R, D), lambda i: (i, 0)),
                  pl.BlockSpec((D, D), lambda i: (0, 0)),
                  pl.BlockSpec((1, D), lambda i: (0, 0)),
                  pl.BlockSpec((1, D), lambda i: (0, 0))],
        out_specs=pl.BlockSpec((_R, D), lambda i: (i, 0)),
        out_shape=jax.ShapeDtypeStruct((N, D), jnp.float32),
    )(parts, parts, root, resid, rW, lg.reshape(1, D), lb.reshape(1, D))


def _tc_pool(h, batch2d):
    def body(h_ref, b_ref, o_ref, s_sum, s_cnt):
        i = pl.program_id(0)

        @pl.when(i == 0)
        def _():
            s_sum[...] = jnp.zeros_like(s_sum)
            s_cnt[...] = jnp.zeros_like(s_cnt)

        oh = (b_ref[...] == lax.broadcasted_iota(jnp.int32, (_R, G), 1)
              ).astype(jnp.float32)
        s_sum[...] += lax.dot_general(oh, h_ref[...],
                                      (((0,), (0,)), ((), ())),
                                      preferred_element_type=jnp.float32)
        s_cnt[...] += lax.dot_general(oh, jnp.ones((_R, 1), jnp.float32),
                                      (((0,), (0,)), ((), ())),
                                      preferred_element_type=jnp.float32)

        @pl.when(i == _NG - 1)
        def _():
            o_ref[...] = s_sum[...] / jnp.maximum(s_cnt[...], 1.0)

    return pl.pallas_call(
        body,
        grid=(_NG,),
        in_specs=[pl.BlockSpec((_R, D), lambda i: (i, 0)),
                  pl.BlockSpec((_R, 1), lambda i: (i, 0))],
        out_specs=pl.BlockSpec((G, D), lambda i: (0, 0)),
        out_shape=jax.ShapeDtypeStruct((G, D), jnp.float32),
        scratch_shapes=[pltpu.VMEM((G, D), jnp.float32),
                        pltpu.VMEM((G, 1), jnp.float32)],
    )(h, batch2d)


def kernel(x, edge_index, batch, proj_W0, proj_b0, proj_W1, proj_b1,
           rel_W0, rel_b0, root_W0, ln_g0, ln_b0,
           rel_W1, rel_b1, root_W1, ln_g1, ln_b1):
    x = x.astype(jnp.float32)
    src, dst = edge_index[0], edge_index[1]
    pad = NW * EPW_PAD - E
    src_p = jnp.concatenate([src, jnp.zeros((pad,), jnp.int32)]
                            ).reshape(NW, NCH, CH)
    dst_p = jnp.concatenate([dst, jnp.full((pad,), N, jnp.int32)]
                            ).reshape(NW, NCH, CH)
    zrows = jnp.zeros((ROWS_PT, D), jnp.float32)

    h = _tc_proj(x, proj_W0, proj_b0, proj_W1, proj_b1)
    root0 = _tc_root(h, root_W0, rel_b0)
    parts = _sc_agg(h, src_p, dst_p, zrows)
    h1 = _tc_layer(parts, root0, x, rel_W0, ln_g0, ln_b0)
    root1 = _tc_root(h1, root_W1, rel_b1)
    parts = _sc_agg(h1, src_p, dst_p, zrows)
    h2 = _tc_layer(parts, root1, h1, rel_W1, ln_g1, ln_b1)
    return _tc_pool(h2, batch.reshape(N, 1))


# speedup vs baseline: 1.1287x; 1.1076x over previous
import functools

import jax
import jax.numpy as jnp
from jax import lax
from jax.experimental import pallas as pl
from jax.experimental.pallas import tpu as pltpu
from jax.experimental.pallas import tpu_sc as plsc

N = 10000
E = 320000
D = 128
G = 64

NC, NS = 2, 16
NW = NC * NS
CH = 128
NCH0 = 60
NCH1 = 97
NCHM = max(NCH0, NCH1)
ROWS_PT = 632
N_PAD = NS * ROWS_PT

_R = 2000
_NG = N // _R


def _leaky(v):
    return jnp.where(v >= 0, v, 0.01 * v)


def _sc_agg(h, src_p, dst_p, zrows):
    mesh = plsc.VectorSubcoreMesh(
        core_axis_name="c", subcore_axis_name="s",
        num_cores=NC, num_subcores=NS)

    @functools.partial(
        pl.kernel,
        out_type=jax.ShapeDtypeStruct((NC, N_PAD, D), jnp.float32),
        mesh=mesh,
        scratch_types=[
            pltpu.VMEM((NCHM, CH), jnp.int32),
            pltpu.VMEM((NCHM, CH), jnp.int32),
            pltpu.VMEM((CH, D), jnp.float32),
            pltpu.VMEM_SHARED((N_PAD, D), jnp.float32),
            pltpu.SemaphoreType.DMA,
        ],
    )
    def k(h_hbm, src_hbm, dst_hbm, z_hbm, out_hbm, sidx, didx, rows, agg, sem):
        c = lax.axis_index("c")
        s = lax.axis_index("s")
        pltpu.sync_copy(z_hbm, agg.at[pl.ds(s * ROWS_PT, ROWS_PT)])
        pltpu.sync_copy(src_hbm.at[c].at[s], sidx)
        pltpu.sync_copy(dst_hbm.at[c].at[s], didx)
        plsc.subcore_barrier()

        def body(j, carry):
            pltpu.async_copy(h_hbm.at[sidx.at[j]], rows, sem).wait()
            pltpu.sync_copy(rows, agg.at[didx.at[j]], add=True)
            return carry

        lax.fori_loop(0, jnp.where(c == 0, NCH0, NCH1), body, 0)
        plsc.subcore_barrier()
        pltpu.sync_copy(agg.at[pl.ds(s * ROWS_PT, ROWS_PT)],
                        out_hbm.at[c].at[pl.ds(s * ROWS_PT, ROWS_PT)])

    return k(h, src_p, dst_p, zrows)


def _tc_proj(x, W0, b0, W1, b1):
    def body(x_ref, w0_ref, b0_ref, w1_ref, b1_ref, o_ref):
        h = _leaky(lax.dot_general(x_ref[...], w0_ref[...],
                                   (((1,), (1,)), ((), ())),
                                   preferred_element_type=jnp.float32)
                   + b0_ref[...])
        o_ref[...] = _leaky(lax.dot_general(h, w1_ref[...],
                                            (((1,), (1,)), ((), ())),
                                            preferred_element_type=jnp.float32)
                            + b1_ref[...])

    return pl.pallas_call(
        body,
        grid=(_NG,),
        in_specs=[pl.BlockSpec((_R, D), lambda i: (i, 0)),
                  pl.BlockSpec((D, D), lambda i: (0, 0)),
                  pl.BlockSpec((1, D), lambda i: (0, 0)),
                  pl.BlockSpec((D, D), lambda i: (0, 0)),
                  pl.BlockSpec((1, D), lambda i: (0, 0))],
        out_specs=pl.BlockSpec((_R, D), lambda i: (i, 0)),
        out_shape=jax.ShapeDtypeStruct((N, D), jnp.float32),
    )(x, W0, b0.reshape(1, D), W1, b1.reshape(1, D))


def _tc_root(h, roW, rb):
    def body(h_ref, w_ref, b_ref, o_ref):
        o_ref[...] = lax.dot_general(h_ref[...], w_ref[...],
                                     (((1,), (1,)), ((), ())),
                                     preferred_element_type=jnp.float32
                                     ) + b_ref[...]

    return pl.pallas_call(
        body,
        grid=(_NG,),
        in_specs=[pl.BlockSpec((_R, D), lambda i: (i, 0)),
                  pl.BlockSpec((D, D), lambda i: (0, 0)),
                  pl.BlockSpec((1, D), lambda i: (0, 0))],
        out_specs=pl.BlockSpec((_R, D), lambda i: (i, 0)),
        out_shape=jax.ShapeDtypeStruct((N, D), jnp.float32),
    )(h, roW, rb.reshape(1, D))


def _tc_layer(parts, root, resid, rW, lg, lb):
    def body(a_ref, b_ref, rt_ref, r_ref, rw_ref, lg_ref, lb_ref, o_ref):
        agg = a_ref[0] + b_ref[0]
        new = lax.dot_general(agg, rw_ref[...], (((1,), (1,)), ((), ())),
                              preferred_element_type=jnp.float32) + rt_ref[...]
        mu = jnp.mean(new, axis=-1, keepdims=True)
        var = jnp.mean((new - mu) ** 2, axis=-1, keepdims=True)
        new = (new - mu) * lax.rsqrt(var + 1e-5) * lg_ref[...] + lb_ref[...]
        o_ref[...] = _leaky(new) + r_ref[...]

    return pl.pallas_call(
        body,
        grid=(_NG,),
        in_specs=[pl.BlockSpec((1, _R, D), lambda i: (0, i, 0)),
                  pl.BlockSpec((1, _R, D), lambda i: (1, i, 0)),
                  pl.BlockSpec((_R, D), lambda i: (i, 0)),
                  pl.BlockSpec((_R, D), lambda i: (i, 0)),
                  pl.BlockSpec((D, D), lambda i: (0, 0)),
                  pl.BlockSpec((1, D), lambda i: (0, 0)),
                  pl.BlockSpec((1, D), lambda i: (0, 0))],
        out_specs=pl.BlockSpec((_R, D), lambda i: (i, 0)),
        out_shape=jax.ShapeDtypeStruct((N, D), jnp.float32),
    )(parts, parts, root, resid, rW, lg.reshape(1, D), lb.reshape(1, D))


def _tc_pool(h, batch2d):
    def body(h_ref, b_ref, o_ref, s_sum, s_cnt):
        i = pl.program_id(0)

        @pl.when(i == 0)
        def _():
            s_sum[...] = jnp.zeros_like(s_sum)
            s_cnt[...] = jnp.zeros_like(s_cnt)

        oh = (b_ref[...] == lax.broadcasted_iota(jnp.int32, (_R, G), 1)
              ).astype(jnp.float32)
        s_sum[...] += lax.dot_general(oh, h_ref[...],
                                      (((0,), (0,)), ((), ())),
                                      preferred_element_type=jnp.float32)
        s_cnt[...] += lax.dot_general(oh, jnp.ones((_R, 1), jnp.float32),
                                      (((0,), (0,)), ((), ())),
                                      preferred_element_type=jnp.float32)

        @pl.when(i == _NG - 1)
        def _():
            o_ref[...] = s_sum[...] / jnp.maximum(s_cnt[...], 1.0)

    return pl.pallas_call(
        body,
        grid=(_NG,),
        in_specs=[pl.BlockSpec((_R, D), lambda i: (i, 0)),
                  pl.BlockSpec((_R, 1), lambda i: (i, 0))],
        out_specs=pl.BlockSpec((G, D), lambda i: (0, 0)),
        out_shape=jax.ShapeDtypeStruct((G, D), jnp.float32),
        scratch_shapes=[pltpu.VMEM((G, D), jnp.float32),
                        pltpu.VMEM((G, 1), jnp.float32)],
    )(h, batch2d)


def kernel(x, edge_index, batch, proj_W0, proj_b0, proj_W1, proj_b1,
           rel_W0, rel_b0, root_W0, ln_g0, ln_b0,
           rel_W1, rel_b1, root_W1, ln_g1, ln_b1):
    x = x.astype(jnp.float32)
    src, dst = edge_index[0], edge_index[1]
    e0 = NS * NCH0 * CH
    pad1 = NS * NCH1 * CH - (E - e0)

    def _part(v, fill):
        p0 = v[:e0].reshape(NS, NCH0, CH)
        p0 = jnp.pad(p0, ((0, 0), (0, NCHM - NCH0), (0, 0)),
                     constant_values=fill)
        p1 = jnp.concatenate([v[e0:], jnp.full((pad1,), fill, jnp.int32)]
                             ).reshape(NS, NCH1, CH)
        p1 = jnp.pad(p1, ((0, 0), (0, NCHM - NCH1), (0, 0)),
                     constant_values=fill)
        return jnp.stack([p0, p1])

    src_p = _part(src, 0)
    dst_p = _part(dst, N)
    zrows = jnp.zeros((ROWS_PT, D), jnp.float32)

    h = _tc_proj(x, proj_W0, proj_b0, proj_W1, proj_b1)
    root0 = _tc_root(h, root_W0, rel_b0)
    parts = _sc_agg(h, src_p, dst_p, zrows)
    h1 = _tc_layer(parts, root0, x, rel_W0, ln_g0, ln_b0)
    root1 = _tc_root(h1, root_W1, rel_b1)
    parts = _sc_agg(h1, src_p, dst_p, zrows)
    h2 = _tc_layer(parts, root1, h1, rel_W1, ln_g1, ln_b1)
    return _tc_pool(h2, batch.reshape(N, 1))


# speedup vs baseline: 1.1463x; 1.0157x over previous
import functools

import jax
import jax.numpy as jnp
from jax import lax
from jax.experimental import pallas as pl
from jax.experimental.pallas import tpu as pltpu
from jax.experimental.pallas import tpu_sc as plsc

N = 10000
E = 320000
D = 128
G = 64

NC, NS = 2, 16
NW = NC * NS
CH = 128
NCH0 = 60
NCH1 = 97
NCHM = max(NCH0, NCH1)
ROWS_PT = 632
N_PAD = NS * ROWS_PT

_R = 2000
_NG = N // _R


def _leaky(v):
    return jnp.where(v >= 0, v, 0.01 * v)


def _sc_agg(h, src_p, dst_p, zrows):
    mesh = plsc.VectorSubcoreMesh(
        core_axis_name="c", subcore_axis_name="s",
        num_cores=NC, num_subcores=NS)

    @functools.partial(
        pl.kernel,
        out_type=jax.ShapeDtypeStruct((NC, N_PAD, D), jnp.float32),
        mesh=mesh,
        scratch_types=[
            pltpu.VMEM((NCHM, CH), jnp.int32),
            pltpu.VMEM((NCHM, CH), jnp.int32),
            pltpu.VMEM((CH, D), jnp.float32),
            pltpu.VMEM_SHARED((N_PAD, D), jnp.float32),
            pltpu.SemaphoreType.DMA,
        ],
    )
    def k(h_hbm, src_hbm, dst_hbm, z_hbm, out_hbm, sidx, didx, rows, agg, sem):
        c = lax.axis_index("c")
        s = lax.axis_index("s")
        pltpu.sync_copy(z_hbm, agg.at[pl.ds(s * ROWS_PT, ROWS_PT)])
        pltpu.sync_copy(src_hbm.at[c].at[s], sidx)
        pltpu.sync_copy(dst_hbm.at[c].at[s], didx)
        plsc.subcore_barrier()

        def body(j, carry):
            pltpu.async_copy(h_hbm.at[sidx.at[j]], rows, sem).wait()
            pltpu.sync_copy(rows, agg.at[didx.at[j]], add=True)
            return carry

        lax.fori_loop(0, jnp.where(c == 0, NCH0, NCH1), body, 0)
        plsc.subcore_barrier()
        pltpu.sync_copy(agg.at[pl.ds(s * ROWS_PT, ROWS_PT)],
                        out_hbm.at[c].at[pl.ds(s * ROWS_PT, ROWS_PT)])

    return k(h, src_p, dst_p, zrows)


def _tc_proj(x, W0, b0, W1, b1):
    def body(x_ref, w0_ref, b0_ref, w1_ref, b1_ref, o_ref):
        h = _leaky(lax.dot_general(x_ref[...], w0_ref[...],
                                   (((1,), (1,)), ((), ())),
                                   preferred_element_type=jnp.float32)
                   + b0_ref[...])
        o_ref[...] = _leaky(lax.dot_general(h, w1_ref[...],
                                            (((1,), (1,)), ((), ())),
                                            preferred_element_type=jnp.float32)
                            + b1_ref[...])

    return pl.pallas_call(
        body,
        grid=(_NG,),
        in_specs=[pl.BlockSpec((_R, D), lambda i: (i, 0)),
                  pl.BlockSpec((D, D), lambda i: (0, 0)),
                  pl.BlockSpec((1, D), lambda i: (0, 0)),
                  pl.BlockSpec((D, D), lambda i: (0, 0)),
                  pl.BlockSpec((1, D), lambda i: (0, 0))],
        out_specs=pl.BlockSpec((_R, D), lambda i: (i, 0)),
        out_shape=jax.ShapeDtypeStruct((N, D), jnp.float32),
    )(x, W0, b0.reshape(1, D), W1, b1.reshape(1, D))


def _tc_root(h, roW, rb):
    def body(h_ref, w_ref, b_ref, o_ref):
        o_ref[...] = lax.dot_general(h_ref[...], w_ref[...],
                                     (((1,), (1,)), ((), ())),
                                     preferred_element_type=jnp.float32
                                     ) + b_ref[...]

    return pl.pallas_call(
        body,
        grid=(_NG,),
        in_specs=[pl.BlockSpec((_R, D), lambda i: (i, 0)),
                  pl.BlockSpec((D, D), lambda i: (0, 0)),
                  pl.BlockSpec((1, D), lambda i: (0, 0))],
        out_specs=pl.BlockSpec((_R, D), lambda i: (i, 0)),
        out_shape=jax.ShapeDtypeStruct((N, D), jnp.float32),
    )(h, roW, rb.reshape(1, D))


def _tc_layer(parts, root, resid, rW, lg, lb):
    def body(a_ref, b_ref, rt_ref, r_ref, rw_ref, lg_ref, lb_ref, o_ref):
        agg = a_ref[0] + b_ref[0]
        new = lax.dot_general(agg, rw_ref[...], (((1,), (1,)), ((), ())),
                              preferred_element_type=jnp.float32) + rt_ref[...]
        mu = jnp.mean(new, axis=-1, keepdims=True)
        var = jnp.mean((new - mu) ** 2, axis=-1, keepdims=True)
        new = (new - mu) * lax.rsqrt(var + 1e-5) * lg_ref[...] + lb_ref[...]
        o_ref[...] = _leaky(new) + r_ref[...]

    return pl.pallas_call(
        body,
        grid=(_NG,),
        in_specs=[pl.BlockSpec((1, _R, D), lambda i: (0, i, 0)),
                  pl.BlockSpec((1, _R, D), lambda i: (1, i, 0)),
                  pl.BlockSpec((_R, D), lambda i: (i, 0)),
                  pl.BlockSpec((_R, D), lambda i: (i, 0)),
                  pl.BlockSpec((D, D), lambda i: (0, 0)),
                  pl.BlockSpec((1, D), lambda i: (0, 0)),
                  pl.BlockSpec((1, D), lambda i: (0, 0))],
        out_specs=pl.BlockSpec((_R, D), lambda i: (i, 0)),
        out_shape=jax.ShapeDtypeStruct((N, D), jnp.float32),
    )(parts, parts, root, resid, rW, lg.reshape(1, D), lb.reshape(1, D))


def _tc_layer_pool(parts, root, resid, rW, lg, lb, batch2d):
    def body(a_ref, b_ref, rt_ref, r_ref, rw_ref, lg_ref, lb_ref, bt_ref,
             o_ref, s_sum, s_cnt):
        i = pl.program_id(0)

        @pl.when(i == 0)
        def _():
            s_sum[...] = jnp.zeros_like(s_sum)
            s_cnt[...] = jnp.zeros_like(s_cnt)

        agg = a_ref[0] + b_ref[0]
        new = lax.dot_general(agg, rw_ref[...], (((1,), (1,)), ((), ())),
                              preferred_element_type=jnp.float32) + rt_ref[...]
        mu = jnp.mean(new, axis=-1, keepdims=True)
        var = jnp.mean((new - mu) ** 2, axis=-1, keepdims=True)
        new = (new - mu) * lax.rsqrt(var + 1e-5) * lg_ref[...] + lb_ref[...]
        h2 = _leaky(new) + r_ref[...]

        oh = (bt_ref[...] == lax.broadcasted_iota(jnp.int32, (_R, G), 1)
              ).astype(jnp.float32)
        s_sum[...] += lax.dot_general(oh, h2, (((0,), (0,)), ((), ())),
                                      preferred_element_type=jnp.float32)
        s_cnt[...] += lax.dot_general(oh, jnp.ones((_R, 1), jnp.float32),
                                      (((0,), (0,)), ((), ())),
                                      preferred_element_type=jnp.float32)

        @pl.when(i == _NG - 1)
        def _():
            o_ref[...] = s_sum[...] / jnp.maximum(s_cnt[...], 1.0)

    return pl.pallas_call(
        body,
        grid=(_NG,),
        in_specs=[pl.BlockSpec((1, _R, D), lambda i: (0, i, 0)),
                  pl.BlockSpec((1, _R, D), lambda i: (1, i, 0)),
                  pl.BlockSpec((_R, D), lambda i: (i, 0)),
                  pl.BlockSpec((_R, D), lambda i: (i, 0)),
                  pl.BlockSpec((D, D), lambda i: (0, 0)),
                  pl.BlockSpec((1, D), lambda i: (0, 0)),
                  pl.BlockSpec((1, D), lambda i: (0, 0)),
                  pl.BlockSpec((_R, 1), lambda i: (i, 0))],
        out_specs=pl.BlockSpec((G, D), lambda i: (0, 0)),
        out_shape=jax.ShapeDtypeStruct((G, D), jnp.float32),
        scratch_shapes=[pltpu.VMEM((G, D), jnp.float32),
                        pltpu.VMEM((G, 1), jnp.float32)],
    )(parts, parts, root, resid, rW, lg.reshape(1, D), lb.reshape(1, D),
      batch2d)


def kernel(x, edge_index, batch, proj_W0, proj_b0, proj_W1, proj_b1,
           rel_W0, rel_b0, root_W0, ln_g0, ln_b0,
           rel_W1, rel_b1, root_W1, ln_g1, ln_b1):
    x = x.astype(jnp.float32)
    src, dst = edge_index[0], edge_index[1]
    e0 = NS * NCH0 * CH
    pad1 = NS * NCH1 * CH - (E - e0)

    def _part(v, fill):
        p0 = v[:e0].reshape(NS, NCH0, CH)
        p0 = jnp.pad(p0, ((0, 0), (0, NCHM - NCH0), (0, 0)),
                     constant_values=fill)
        p1 = jnp.concatenate([v[e0:], jnp.full((pad1,), fill, jnp.int32)]
                             ).reshape(NS, NCH1, CH)
        p1 = jnp.pad(p1, ((0, 0), (0, NCHM - NCH1), (0, 0)),
                     constant_values=fill)
        return jnp.stack([p0, p1])

    src_p = _part(src, 0)
    dst_p = _part(dst, N)
    zrows = jnp.zeros((ROWS_PT, D), jnp.float32)

    h = _tc_proj(x, proj_W0, proj_b0, proj_W1, proj_b1)
    root0 = _tc_root(h, root_W0, rel_b0)
    parts = _sc_agg(h, src_p, dst_p, zrows)
    h1 = _tc_layer(parts, root0, x, rel_W0, ln_g0, ln_b0)
    root1 = _tc_root(h1, root_W1, rel_b1)
    parts = _sc_agg(h1, src_p, dst_p, zrows)
    return _tc_layer_pool(parts, root1, h1, rel_W1, ln_g1, ln_b1,
                          batch.reshape(N, 1))
